# baseline (device time: 488453 ns/iter reference)
import os

import jax
import jax.numpy as jnp
from jax import lax
from jax.experimental import pallas as pl
from jax.experimental.pallas import tpu as pltpu

_SKIP_RING = bool(int(os.environ.get("SKIP_RING", "0")))

N_DEV = 16
T = 1024
D = 2048
VSH = 16384
TR = 256
TC = 8192
HTC = TC // 2
S = 4
KC = 1024
N_CHUNKS = TC // KC


def _ring_coords(q):
    xq = q // 8
    hq = q // 2
    zq = jnp.where(xq == 0, hq, 7 - hq)
    yq = ((q + 1) // 2) % 2
    return xq, yq, zq


def _tile_offsets(o):
    xo, yo, zo = _ring_coords(o)
    return zo * TR, (yo * 2 + xo) * TC


def kernel(x, W):
    def body(x_hbm, w_hbm, out_hbm, xs, wbuf, logits, stage_r, stage_l,
             stats_s, stats_r, comm_r, comm_l, load_sems, out_sems,
             st_send, st_recv, sr_send, sr_recv, sl_send, sl_recv,
             cred_r, cred_l):
        mx = lax.axis_index("x")
        my = lax.axis_index("y")
        mz = lax.axis_index("z")
        p = jnp.where(
            mx == 0,
            jnp.where(my == mz % 2, 2 * mz, 2 * mz + 1),
            jnp.where(my == mz % 2, 15 - 2 * mz, 14 - 2 * mz),
        )
        right = _ring_coords((p + 1) % N_DEV)
        left = _ring_coords((p - 1) % N_DEV)

        cpx = pltpu.make_async_copy(
            x_hbm.at[pl.ds(mz * TR, TR), :], xs, st_send.at[0])
        cpx.start()
        wc0 = mx * TC

        def w_copy(c):
            return pltpu.make_async_copy(
                w_hbm.at[:, pl.ds(wc0 + c * KC, KC)], wbuf.at[c % 2],
                load_sems.at[c % 2])

        cpw = w_copy(0)
        cpw.start()
        cpx.wait()
        xb = xs[:, :].astype(jnp.bfloat16)
        s = jnp.zeros((TR, 1), jnp.float32)
        for c in range(N_CHUNKS):
            cur = cpw
            if c + 1 < N_CHUNKS:
                cpw = w_copy(c + 1)
                cpw.start()
            cur.wait()
            wb = wbuf[c % 2, :, :].astype(jnp.bfloat16)
            lc = jnp.dot(xb, wb, preferred_element_type=jnp.float32)
            logits[:, c * KC:(c + 1) * KC] = lc
            s = s + jnp.sum(jnp.exp(lc), axis=1, keepdims=True)

        bsem = pltpu.get_barrier_semaphore()
        for nbr in (left, right, (1 - mx, my, mz)):
            pl.semaphore_signal(
                bsem, inc=1, device_id=nbr,
                device_id_type=pl.DeviceIdType.MESH,
            )
        pl.semaphore_wait(bsem, 3)

        stats_s[:, 0:128] = jnp.broadcast_to(s, (TR, 128))
        for step, part in enumerate(((mx, 1 - my, mz), (1 - mx, my, mz))):
            st = pltpu.make_async_remote_copy(
                src_ref=stats_s.at[:, pl.ds(step * 128, 128)],
                dst_ref=stats_r.at[:, pl.ds(step * 128, 128)],
                send_sem=st_send.at[step], recv_sem=st_recv.at[step],
                device_id=part, device_id_type=pl.DeviceIdType.MESH,
            )
            st.start()
            st.wait()
            blk = slice(step * 128, step * 128 + 128)
            stats_s[:, (step + 1) * 128:(step + 2) * 128] = (
                stats_s[:, blk] + stats_r[:, blk])
        gs = stats_s[:, 256:257]

        t32 = jnp.exp(logits[:, :]) / gs
        comm_r[0, :, :] = t32[:, :HTC].astype(jnp.bfloat16)
        comm_l[0, :, :] = t32[:, HTC:].astype(jnp.bfloat16)
        stage_r[:, :] = t32[:, :HTC]
        stage_l[:, :] = t32[:, HTC:]
        my_ro, my_co = _tile_offsets(p)
        out_dma = {}
        for dirn, stage, osl, coff in (
                ("r", stage_r, 0, 0), ("l", stage_l, 1, HTC)):
            oc = pltpu.make_async_copy(
                stage,
                out_hbm.at[pl.ds(my_ro, TR), pl.ds(my_co + coff, HTC)],
                out_sems.at[osl])
            oc.start()
            out_dma[dirn] = oc

        def process(dirn, slot, origin):
            stage, comm, osl, coff = {
                "r": (stage_r, comm_r, 0, 0),
                "l": (stage_l, comm_l, 1, HTC),
            }[dirn]
            out_dma[dirn].wait()
            stage[:, :] = comm[slot, :, :].astype(jnp.float32)
            oro, oco = _tile_offsets(origin)
            oc = pltpu.make_async_copy(
                stage,
                out_hbm.at[pl.ds(oro, TR), pl.ds(oco + coff, HTC)],
                out_sems.at[osl])
            oc.start()
            out_dma[dirn] = oc

        n_hops = 0 if _SKIP_RING else N_DEV - 1
        for h in range(n_hops):
            if h >= 3:
                pl.semaphore_wait(cred_r, 1)
                pl.semaphore_wait(cred_l, 1)
            rdma_r = pltpu.make_async_remote_copy(
                src_ref=comm_r.at[h % S], dst_ref=comm_r.at[(h + 1) % S],
                send_sem=sr_send.at[h], recv_sem=sr_recv.at[h],
                device_id=right, device_id_type=pl.DeviceIdType.MESH,
            )
            rdma_l = pltpu.make_async_remote_copy(
                src_ref=comm_l.at[h % S], dst_ref=comm_l.at[(h + 1) % S],
                send_sem=sl_send.at[h], recv_sem=sl_recv.at[h],
                device_id=left, device_id_type=pl.DeviceIdType.MESH,
            )
            rdma_r.start()
            rdma_l.start()
            if h >= 1:
                process("r", h % S, (p - h) % N_DEV)
                process("l", h % S, (p + h) % N_DEV)
            rdma_r.wait()
            rdma_l.wait()
            if h <= 11:
                pl.semaphore_signal(
                    cred_r, inc=1, device_id=left,
                    device_id_type=pl.DeviceIdType.MESH)
                pl.semaphore_signal(
                    cred_l, inc=1, device_id=right,
                    device_id_type=pl.DeviceIdType.MESH)

        if not _SKIP_RING:
            process("r", (N_DEV - 1) % S, (p - (N_DEV - 1)) % N_DEV)
            process("l", (N_DEV - 1) % S, (p + (N_DEV - 1)) % N_DEV)
        out_dma["r"].wait()
        out_dma["l"].wait()

    return pl.pallas_call(
        body,
        out_shape=jax.ShapeDtypeStruct((T, 2 * VSH), jnp.float32),
        in_specs=[
            pl.BlockSpec(memory_space=pl.ANY),
            pl.BlockSpec(memory_space=pl.ANY),
        ],
        out_specs=pl.BlockSpec(memory_space=pl.ANY),
        scratch_shapes=[
            pltpu.VMEM((TR, D), jnp.float32),
            pltpu.VMEM((2, D, KC), jnp.float32),
            pltpu.VMEM((TR, TC), jnp.float32),
            pltpu.VMEM((TR, HTC), jnp.float32),
            pltpu.VMEM((TR, HTC), jnp.float32),
            pltpu.VMEM((TR, 384), jnp.float32),
            pltpu.VMEM((TR, 384), jnp.float32),
            pltpu.VMEM((S, TR, HTC), jnp.bfloat16),
            pltpu.VMEM((S, TR, HTC), jnp.bfloat16),
            pltpu.SemaphoreType.DMA((2,)),
            pltpu.SemaphoreType.DMA((2,)),
            pltpu.SemaphoreType.DMA((2,)),
            pltpu.SemaphoreType.DMA((2,)),
            pltpu.SemaphoreType.DMA((N_DEV - 1,)),
            pltpu.SemaphoreType.DMA((N_DEV - 1,)),
            pltpu.SemaphoreType.DMA((N_DEV - 1,)),
            pltpu.SemaphoreType.DMA((N_DEV - 1,)),
            pltpu.SemaphoreType.REGULAR,
            pltpu.SemaphoreType.REGULAR,
        ],
        compiler_params=pltpu.CompilerParams(
            collective_id=0, vmem_limit_bytes=64 * 1024 * 1024),
    )(x, W)


# device time: 488133 ns/iter; 1.0007x vs baseline; 1.0007x over previous
import os

import jax
import jax.numpy as jnp
from jax import lax
from jax.experimental import pallas as pl
from jax.experimental.pallas import tpu as pltpu

_SKIP_RING = bool(int(os.environ.get("SKIP_RING", "0")))

N_DEV = 16
T = 1024
D = 2048
VSH = 16384
TR = 256
TC = 8192
HTC = TC // 2
S = 4
KC = 1024
N_CHUNKS = TC // KC


def _ring_coords(q):
    xq = q // 8
    hq = q // 2
    zq = jnp.where(xq == 0, hq, 7 - hq)
    yq = ((q + 1) // 2) % 2
    return xq, yq, zq


def _tile_offsets(o):
    xo, yo, zo = _ring_coords(o)
    return zo * TR, (yo * 2 + xo) * TC


def kernel(x, W):
    def body(x_hbm, w_hbm, out_hbm, xs, xbb, wbuf, evals, stage_r, stage_l,
             stats_s, stats_r, comm_r, comm_l, load_sems, out_sems,
             st_send, st_recv, sr_send, sr_recv, sl_send, sl_recv,
             cred_r, cred_l):
        mx = lax.axis_index("x")
        my = lax.axis_index("y")
        mz = lax.axis_index("z")
        p = jnp.where(
            mx == 0,
            jnp.where(my == mz % 2, 2 * mz, 2 * mz + 1),
            jnp.where(my == mz % 2, 15 - 2 * mz, 14 - 2 * mz),
        )
        right = _ring_coords((p + 1) % N_DEV)
        left = _ring_coords((p - 1) % N_DEV)

        cpx = pltpu.make_async_copy(
            x_hbm.at[pl.ds(mz * TR, TR), :], xs, st_send.at[0])
        cpx.start()
        wc0 = mx * TC

        def w_copy(c):
            return pltpu.make_async_copy(
                w_hbm.at[:, pl.ds(wc0 + c * KC, KC)], wbuf.at[c % 2],
                load_sems.at[c % 2])

        cpw = w_copy(0)
        cpw.start()
        cpx.wait()
        xbb[:, :] = xs[:, :].astype(jnp.bfloat16)
        xb = xbb[:, :]
        s = jnp.zeros((TR, 1), jnp.float32)
        for c in range(N_CHUNKS):
            cur = cpw
            if c + 1 < N_CHUNKS:
                cpw = w_copy(c + 1)
                cpw.start()
            cur.wait()
            wb = wbuf[c % 2, :, :].astype(jnp.bfloat16)
            lc = jnp.dot(xb, wb, preferred_element_type=jnp.float32)
            ec = jnp.exp(lc)
            evals[:, c * KC:(c + 1) * KC] = ec
            s = s + jnp.sum(ec, axis=1, keepdims=True)

        bsem = pltpu.get_barrier_semaphore()
        for nbr in (left, right, (1 - mx, my, mz)):
            pl.semaphore_signal(
                bsem, inc=1, device_id=nbr,
                device_id_type=pl.DeviceIdType.MESH,
            )
        pl.semaphore_wait(bsem, 3)

        stats_s[:, 0:128] = jnp.broadcast_to(s, (TR, 128))
        for step, part in enumerate(((mx, 1 - my, mz), (1 - mx, my, mz))):
            st = pltpu.make_async_remote_copy(
                src_ref=stats_s.at[:, pl.ds(step * 128, 128)],
                dst_ref=stats_r.at[:, pl.ds(step * 128, 128)],
                send_sem=st_send.at[step], recv_sem=st_recv.at[step],
                device_id=part, device_id_type=pl.DeviceIdType.MESH,
            )
            st.start()
            st.wait()
            blk = slice(step * 128, step * 128 + 128)
            stats_s[:, (step + 1) * 128:(step + 2) * 128] = (
                stats_s[:, blk] + stats_r[:, blk])
        gs = stats_s[:, 256:257]

        inv = 1.0 / gs
        t32 = evals[:, :] * inv
        comm_r[0, :, :] = t32[:, :HTC].astype(jnp.bfloat16)
        comm_l[0, :, :] = t32[:, HTC:].astype(jnp.bfloat16)
        my_ro, my_co = _tile_offsets(p)
        out_dma = {}

        def stage_own():
            stage_r[:, :] = t32[:, :HTC]
            stage_l[:, :] = t32[:, HTC:]
            for dirn, stage, osl, coff in (
                    ("r", stage_r, 0, 0), ("l", stage_l, 1, HTC)):
                oc = pltpu.make_async_copy(
                    stage,
                    out_hbm.at[pl.ds(my_ro, TR), pl.ds(my_co + coff, HTC)],
                    out_sems.at[osl])
                oc.start()
                out_dma[dirn] = oc

        def process(dirn, slot, origin):
            stage, comm, osl, coff = {
                "r": (stage_r, comm_r, 0, 0),
                "l": (stage_l, comm_l, 1, HTC),
            }[dirn]
            out_dma[dirn].wait()
            stage[:, :] = comm[slot, :, :].astype(jnp.float32)
            oro, oco = _tile_offsets(origin)
            oc = pltpu.make_async_copy(
                stage,
                out_hbm.at[pl.ds(oro, TR), pl.ds(oco + coff, HTC)],
                out_sems.at[osl])
            oc.start()
            out_dma[dirn] = oc

        n_hops = 0 if _SKIP_RING else N_DEV - 1
        for h in range(n_hops):
            if h >= 3:
                pl.semaphore_wait(cred_r, 1)
                pl.semaphore_wait(cred_l, 1)
            rdma_r = pltpu.make_async_remote_copy(
                src_ref=comm_r.at[h % S], dst_ref=comm_r.at[(h + 1) % S],
                send_sem=sr_send.at[h], recv_sem=sr_recv.at[h],
                device_id=right, device_id_type=pl.DeviceIdType.MESH,
            )
            rdma_l = pltpu.make_async_remote_copy(
                src_ref=comm_l.at[h % S], dst_ref=comm_l.at[(h + 1) % S],
                send_sem=sl_send.at[h], recv_sem=sl_recv.at[h],
                device_id=left, device_id_type=pl.DeviceIdType.MESH,
            )
            rdma_r.start()
            rdma_l.start()
            if h == 0:
                stage_own()
            else:
                process("r", h % S, (p - h) % N_DEV)
                process("l", h % S, (p + h) % N_DEV)
            rdma_r.wait()
            rdma_l.wait()
            if h <= 11:
                pl.semaphore_signal(
                    cred_r, inc=1, device_id=left,
                    device_id_type=pl.DeviceIdType.MESH)
                pl.semaphore_signal(
                    cred_l, inc=1, device_id=right,
                    device_id_type=pl.DeviceIdType.MESH)

        if _SKIP_RING:
            stage_own()
        else:
            process("r", (N_DEV - 1) % S, (p - (N_DEV - 1)) % N_DEV)
            process("l", (N_DEV - 1) % S, (p + (N_DEV - 1)) % N_DEV)
        out_dma["r"].wait()
        out_dma["l"].wait()

    return pl.pallas_call(
        body,
        out_shape=jax.ShapeDtypeStruct((T, 2 * VSH), jnp.float32),
        in_specs=[
            pl.BlockSpec(memory_space=pl.ANY),
            pl.BlockSpec(memory_space=pl.ANY),
        ],
        out_specs=pl.BlockSpec(memory_space=pl.ANY),
        scratch_shapes=[
            pltpu.VMEM((TR, D), jnp.float32),
            pltpu.VMEM((TR, D), jnp.bfloat16),
            pltpu.VMEM((2, D, KC), jnp.float32),
            pltpu.VMEM((TR, TC), jnp.float32),
            pltpu.VMEM((TR, HTC), jnp.float32),
            pltpu.VMEM((TR, HTC), jnp.float32),
            pltpu.VMEM((TR, 384), jnp.float32),
            pltpu.VMEM((TR, 384), jnp.float32),
            pltpu.VMEM((S, TR, HTC), jnp.bfloat16),
            pltpu.VMEM((S, TR, HTC), jnp.bfloat16),
            pltpu.SemaphoreType.DMA((2,)),
            pltpu.SemaphoreType.DMA((2,)),
            pltpu.SemaphoreType.DMA((2,)),
            pltpu.SemaphoreType.DMA((2,)),
            pltpu.SemaphoreType.DMA((N_DEV - 1,)),
            pltpu.SemaphoreType.DMA((N_DEV - 1,)),
            pltpu.SemaphoreType.DMA((N_DEV - 1,)),
            pltpu.SemaphoreType.DMA((N_DEV - 1,)),
            pltpu.SemaphoreType.REGULAR,
            pltpu.SemaphoreType.REGULAR,
        ],
        compiler_params=pltpu.CompilerParams(
            collective_id=0, vmem_limit_bytes=64 * 1024 * 1024),
    )(x, W)


# device time: 463137 ns/iter; 1.0547x vs baseline; 1.0540x over previous
import os

import jax
import jax.numpy as jnp
from jax import lax
from jax.experimental import pallas as pl
from jax.experimental.pallas import tpu as pltpu

_SKIP_RING = bool(int(os.environ.get("SKIP_RING", "0")))

N_DEV = 16
T = 1024
D = 2048
VSH = 16384
TR = 256
TC = 8192
HTC = TC // 2
QW = HTC // 2
S = 4
KC = 1024
N_CHUNKS = TC // KC


def _ring_coords(q):
    xq = q // 8
    hq = q // 2
    zq = jnp.where(xq == 0, hq, 7 - hq)
    yq = ((q + 1) // 2) % 2
    return xq, yq, zq


def _tile_offsets(o):
    xo, yo, zo = _ring_coords(o)
    return zo * TR, (yo * 2 + xo) * TC


def kernel(x, W):
    def body(x_hbm, w_hbm, out_hbm, xs, xbb, wbuf, evals,
             stage_r0, stage_r1, stage_l0, stage_l1, stats_s, stats_r,
             comm_r0, comm_r1, comm_l0, comm_l1, load_sems, out_sems,
             st_send, st_recv, sr0_send, sr0_recv, sr1_send, sr1_recv,
             sl0_send, sl0_recv, sl1_send, sl1_recv,
             cred_r0, cred_r1, cred_l0, cred_l1):
        mx = lax.axis_index("x")
        my = lax.axis_index("y")
        mz = lax.axis_index("z")
        p = jnp.where(
            mx == 0,
            jnp.where(my == mz % 2, 2 * mz, 2 * mz + 1),
            jnp.where(my == mz % 2, 15 - 2 * mz, 14 - 2 * mz),
        )
        right = _ring_coords((p + 1) % N_DEV)
        left = _ring_coords((p - 1) % N_DEV)

        cpx = pltpu.make_async_copy(
            x_hbm.at[pl.ds(mz * TR, TR), :], xs, st_send.at[0])
        cpx.start()
        wc0 = mx * TC

        def w_copy(c):
            return pltpu.make_async_copy(
                w_hbm.at[:, pl.ds(wc0 + c * KC, KC)], wbuf.at[c % 2],
                load_sems.at[c % 2])

        cpw = w_copy(0)
        cpw.start()
        cpx.wait()
        xbb[:, :] = xs[:, :].astype(jnp.bfloat16)
        xb = xbb[:, :]
        s = jnp.zeros((TR, 1), jnp.float32)
        for c in range(N_CHUNKS):
            cur = cpw
            if c + 1 < N_CHUNKS:
                cpw = w_copy(c + 1)
                cpw.start()
            cur.wait()
            wb = wbuf[c % 2, :, :].astype(jnp.bfloat16)
            lc = jnp.dot(xb, wb, preferred_element_type=jnp.float32)
            ec = jnp.exp(lc)
            evals[:, c * KC:(c + 1) * KC] = ec
            s = s + jnp.sum(ec, axis=1, keepdims=True)

        bsem = pltpu.get_barrier_semaphore()
        for nbr in (left, right, (1 - mx, my, mz)):
            pl.semaphore_signal(
                bsem, inc=1, device_id=nbr,
                device_id_type=pl.DeviceIdType.MESH,
            )
        pl.semaphore_wait(bsem, 3)

        stats_s[:, 0:128] = jnp.broadcast_to(s, (TR, 128))
        for step, part in enumerate(((mx, 1 - my, mz), (1 - mx, my, mz))):
            st = pltpu.make_async_remote_copy(
                src_ref=stats_s.at[:, pl.ds(step * 128, 128)],
                dst_ref=stats_r.at[:, pl.ds(step * 128, 128)],
                send_sem=st_send.at[step], recv_sem=st_recv.at[step],
                device_id=part, device_id_type=pl.DeviceIdType.MESH,
            )
            st.start()
            st.wait()
            blk = slice(step * 128, step * 128 + 128)
            stats_s[:, (step + 1) * 128:(step + 2) * 128] = (
                stats_s[:, blk] + stats_r[:, blk])
        gs = stats_s[:, 256:257]

        inv = 1.0 / gs
        t32 = evals[:, :] * inv
        streams = (
            ("r", 0, comm_r0, stage_r0, sr0_send, sr0_recv, cred_r0, 0),
            ("l", 0, comm_l0, stage_l0, sl0_send, sl0_recv, cred_l0, HTC),
            ("r", 1, comm_r1, stage_r1, sr1_send, sr1_recv, cred_r1, QW),
            ("l", 1, comm_l1, stage_l1, sl1_send, sl1_recv, cred_l1, HTC + QW),
        )
        for k, (dirn, q, cm, stg, _ss, _rs, _cr, coff) in enumerate(streams):
            cm[0, :, :] = t32[:, coff:coff + QW].astype(jnp.bfloat16)
        my_ro, my_co = _tile_offsets(p)
        out_dma = {}

        def stage_own():
            for k, (dirn, q, cm, stg, _ss, _rs, _cr, coff) in enumerate(streams):
                stg[:, :] = t32[:, coff:coff + QW]
                oc = pltpu.make_async_copy(
                    stg,
                    out_hbm.at[pl.ds(my_ro, TR), pl.ds(my_co + coff, QW)],
                    out_sems.at[k])
                oc.start()
                out_dma[dirn, q] = oc

        def process(stream, k, slot, origin):
            dirn, q, cm, stg, _ss, _rs, _cr, coff = stream
            out_dma[dirn, q].wait()
            stg[:, :] = cm[slot, :, :].astype(jnp.float32)
            oro, oco = _tile_offsets(origin)
            oc = pltpu.make_async_copy(
                stg,
                out_hbm.at[pl.ds(oro, TR), pl.ds(oco + coff, QW)],
                out_sems.at[k])
            oc.start()
            out_dma[dirn, q] = oc

        rds = {}
        n_rounds = 0 if _SKIP_RING else N_DEV
        for h in range(n_rounds):
            for k, stream in enumerate(streams):
                dirn, q, cm, stg, ssm, rsm, crd, coff = stream
                tgt = right if dirn == "r" else left
                if h >= 1:
                    rds[dirn, q].wait()
                if h <= N_DEV - 2:
                    if h >= 3:
                        pl.semaphore_wait(crd, 1)
                    rd = pltpu.make_async_remote_copy(
                        src_ref=cm.at[h % S], dst_ref=cm.at[(h + 1) % S],
                        send_sem=ssm.at[h], recv_sem=rsm.at[h],
                        device_id=tgt, device_id_type=pl.DeviceIdType.MESH,
                    )
                    rd.start()
                    rds[dirn, q] = rd
                if h >= 1:
                    origin = ((p - h) if dirn == "r" else (p + h)) % N_DEV
                    process(stream, k, h % S, origin)
                    if h <= 12:
                        ups = left if dirn == "r" else right
                        pl.semaphore_signal(
                            crd, inc=1, device_id=ups,
                            device_id_type=pl.DeviceIdType.MESH)
            if h == 0:
                stage_own()

        if _SKIP_RING:
            stage_own()
        for key in out_dma:
            out_dma[key].wait()

    return pl.pallas_call(
        body,
        out_shape=jax.ShapeDtypeStruct((T, 2 * VSH), jnp.float32),
        in_specs=[
            pl.BlockSpec(memory_space=pl.ANY),
            pl.BlockSpec(memory_space=pl.ANY),
        ],
        out_specs=pl.BlockSpec(memory_space=pl.ANY),
        scratch_shapes=[
            pltpu.VMEM((TR, D), jnp.float32),
            pltpu.VMEM((TR, D), jnp.bfloat16),
            pltpu.VMEM((2, D, KC), jnp.float32),
            pltpu.VMEM((TR, TC), jnp.float32),
            pltpu.VMEM((TR, QW), jnp.float32),
            pltpu.VMEM((TR, QW), jnp.float32),
            pltpu.VMEM((TR, QW), jnp.float32),
            pltpu.VMEM((TR, QW), jnp.float32),
            pltpu.VMEM((TR, 384), jnp.float32),
            pltpu.VMEM((TR, 384), jnp.float32),
            pltpu.VMEM((S, TR, QW), jnp.bfloat16),
            pltpu.VMEM((S, TR, QW), jnp.bfloat16),
            pltpu.VMEM((S, TR, QW), jnp.bfloat16),
            pltpu.VMEM((S, TR, QW), jnp.bfloat16),
            pltpu.SemaphoreType.DMA((2,)),
            pltpu.SemaphoreType.DMA((4,)),
            pltpu.SemaphoreType.DMA((2,)),
            pltpu.SemaphoreType.DMA((2,)),
            pltpu.SemaphoreType.DMA((N_DEV - 1,)),
            pltpu.SemaphoreType.DMA((N_DEV - 1,)),
            pltpu.SemaphoreType.DMA((N_DEV - 1,)),
            pltpu.SemaphoreType.DMA((N_DEV - 1,)),
            pltpu.SemaphoreType.DMA((N_DEV - 1,)),
            pltpu.SemaphoreType.DMA((N_DEV - 1,)),
            pltpu.SemaphoreType.DMA((N_DEV - 1,)),
            pltpu.SemaphoreType.DMA((N_DEV - 1,)),
            pltpu.SemaphoreType.REGULAR,
            pltpu.SemaphoreType.REGULAR,
            pltpu.SemaphoreType.REGULAR,
            pltpu.SemaphoreType.REGULAR,
        ],
        compiler_params=pltpu.CompilerParams(
            collective_id=0, vmem_limit_bytes=64 * 1024 * 1024),
    )(x, W)
